# Initial kernel scaffold; baseline (speedup 1.0000x reference)
import jax
import jax.numpy as jnp
from jax.experimental import pallas as pl


def kernel(x, edge_index, edge_attr, W1, b1, ln_g, ln_b, W2, b2):
    raise NotImplementedError("write your pallas kernel here")


# speedup vs baseline: 1.1397x; 1.1397x over previous
import functools

import jax
import jax.numpy as jnp
from jax import lax
from jax.experimental import pallas as pl
from jax.experimental.pallas import tpu as pltpu
from jax.experimental.pallas import tpu_sc as plsc

N = 50000
E = 800000
D = 64
H = 128
EPS = 1e-7

NSC = 2
NTILES = 16
NPS = 25024
SROWS = 25088
STRIPE = SROWS // NTILES
TRASH = NPS
NPAD = NSC * NPS
EPT = E // NTILES
CHUNK = 80
NCHUNK = EPT // CHUNK
QC = D // 4
ACC = 2 * QC


def _sc_accumulate():
    mesh = plsc.VectorSubcoreMesh(core_axis_name="c", subcore_axis_name="s")

    @functools.partial(
        pl.kernel,
        out_type=tuple(
            jax.ShapeDtypeStruct((NPAD, ACC), jnp.float32) for _ in range(4)),
        mesh=mesh,
        scratch_types=[
            pltpu.VMEM((CHUNK,), jnp.int32),
            pltpu.VMEM((CHUNK,), jnp.int32),
            pltpu.VMEM((CHUNK,), jnp.int32),
            pltpu.VMEM((CHUNK, QC), jnp.float32),
            pltpu.VMEM((CHUNK, QC), jnp.float32),
            pltpu.VMEM((CHUNK, ACC), jnp.float32),
            pltpu.VMEM((STRIPE, ACC), jnp.float32),
            pltpu.VMEM_SHARED((SROWS, ACC), jnp.float32),
            pltpu.SemaphoreType.DMA,
        ],
        compiler_params=pltpu.CompilerParams(use_tc_tiling_on_sc=False),
    )
    def k(x0_hbm, x1_hbm, x2_hbm, x3_hbm,
          esrc_hbm, edst_hbm,
          ea0_hbm, ea1_hbm, ea2_hbm, ea3_hbm,
          o0_hbm, o1_hbm, o2_hbm, o3_hbm,
          src_v, dst_v, sidx_v, xg_v, ea_v, vb_v, zb_v,
          acc_sh, sem):
        c = lax.axis_index("c")
        s = lax.axis_index("s")
        base_node = c * NPS
        ebase = s * EPT
        xq = [x0_hbm, x1_hbm, x2_hbm, x3_hbm]
        eaq = [ea0_hbm, ea1_hbm, ea2_hbm, ea3_hbm]
        oq = [o0_hbm, o1_hbm, o2_hbm, o3_hbm]

        @pl.loop(0, STRIPE)
        def _(i):
            z = jnp.zeros((16,), jnp.float32)
            zb_v[i, pl.ds(0, 16)] = z
            zb_v[i, pl.ds(16, 16)] = z

        for q in range(4):
            pltpu.sync_copy(zb_v, acc_sh.at[pl.ds(s * STRIPE, STRIPE)])
            plsc.subcore_barrier()

            @pl.loop(0, NCHUNK)
            def _(i):
                base = ebase + i * CHUNK
                pltpu.sync_copy(esrc_hbm.at[pl.ds(base, CHUNK)], src_v)
                pltpu.sync_copy(edst_hbm.at[pl.ds(base, CHUNK)], dst_v)
                gat = pltpu.async_copy(xq[q].at[src_v], xg_v, sem)
                pltpu.sync_copy(eaq[q].at[pl.ds(base, CHUNK)], ea_v)

                @pl.loop(0, CHUNK // 16)
                def _(j):
                    dv = dst_v[pl.ds(j * 16, 16)]
                    ld = dv - base_node
                    ok = (ld >= 0) & (ld < NPS)
                    sidx_v[pl.ds(j * 16, 16)] = jnp.where(ok, ld, TRASH)

                gat.wait()

                @pl.loop(0, CHUNK)
                def _(r):
                    vx = xg_v[r, pl.ds(0, 16)]
                    ve = ea_v[r, pl.ds(0, 16)]
                    m = jnp.maximum(vx + ve, 0.0) + EPS
                    p = jnp.exp(m)
                    vb_v[r, pl.ds(0, 16)] = p
                    vb_v[r, pl.ds(16, 16)] = m * p

                pltpu.sync_copy(vb_v, acc_sh.at[sidx_v], add=True)

            plsc.subcore_barrier()

            gbase = base_node + s * STRIPE

            @pl.when(s < NTILES - 1)
            def _():
                pltpu.sync_copy(acc_sh.at[pl.ds(s * STRIPE, STRIPE)],
                                oq[q].at[pl.ds(gbase, STRIPE)])

            last = NPS - (NTILES - 1) * STRIPE

            @pl.when(s == NTILES - 1)
            def _():
                pltpu.sync_copy(acc_sh.at[pl.ds(s * STRIPE, last)],
                                oq[q].at[pl.ds(gbase, last)])

            plsc.subcore_barrier()

    return k


_ROWS_BLK = 3128


def _mlp_body(a0_ref, a1_ref, a2_ref, a3_ref, x_ref, w1_ref, b1_ref,
              g_ref, bt_ref, w2_ref, b2_ref, o_ref):
    quarters = [a0_ref[...], a1_ref[...], a2_ref[...], a3_ref[...]]
    s_acc = jnp.concatenate([a[:, :QC] for a in quarters], axis=1)
    w_acc = jnp.concatenate([a[:, QC:] for a in quarters], axis=1)
    agg = w_acc / (s_acc + 1e-16)
    out = agg + x_ref[...]
    hpre = jnp.dot(out, w1_ref[...],
                   preferred_element_type=jnp.float32) + b1_ref[...]
    mu = jnp.mean(hpre, axis=-1, keepdims=True)
    var = jnp.mean((hpre - mu) ** 2, axis=-1, keepdims=True)
    hn = (hpre - mu) * lax.rsqrt(var + 1e-5) * g_ref[...] + bt_ref[...]
    hr = jnp.maximum(hn, 0.0)
    o_ref[...] = jnp.dot(hr, w2_ref[...],
                         preferred_element_type=jnp.float32) + b2_ref[...]


def _mlp(accs, x, W1, b1, ln_g, ln_b, W2, b2):
    nblk = pl.cdiv(N, _ROWS_BLK)
    return pl.pallas_call(
        _mlp_body,
        out_shape=jax.ShapeDtypeStruct((N, D), jnp.float32),
        grid=(nblk,),
        in_specs=[
            pl.BlockSpec((_ROWS_BLK, ACC), lambda i: (i, 0)),
            pl.BlockSpec((_ROWS_BLK, ACC), lambda i: (i, 0)),
            pl.BlockSpec((_ROWS_BLK, ACC), lambda i: (i, 0)),
            pl.BlockSpec((_ROWS_BLK, ACC), lambda i: (i, 0)),
            pl.BlockSpec((_ROWS_BLK, D), lambda i: (i, 0)),
            pl.BlockSpec((D, H), lambda i: (0, 0)),
            pl.BlockSpec((1, H), lambda i: (0, 0)),
            pl.BlockSpec((1, H), lambda i: (0, 0)),
            pl.BlockSpec((1, H), lambda i: (0, 0)),
            pl.BlockSpec((H, D), lambda i: (0, 0)),
            pl.BlockSpec((1, D), lambda i: (0, 0)),
        ],
        out_specs=pl.BlockSpec((_ROWS_BLK, D), lambda i: (i, 0)),
    )(*accs, x, W1, b1, ln_g, ln_b, W2, b2)


def kernel(x, edge_index, edge_attr, W1, b1, ln_g, ln_b, W2, b2):
    xqs = [x[:, i * QC:(i + 1) * QC] for i in range(4)]
    eaqs = [edge_attr[:, i * QC:(i + 1) * QC] for i in range(4)]
    accs = _sc_accumulate()(*xqs, edge_index[0], edge_index[1], *eaqs)
    accs = [a[:N] for a in accs]
    return _mlp(accs, x,
                W1, b1.reshape(1, H), ln_g.reshape(1, H),
                ln_b.reshape(1, H), W2, b2.reshape(1, D))


# speedup vs baseline: 1.8955x; 1.6631x over previous
import functools

import jax
import jax.numpy as jnp
from jax import lax
from jax.experimental import pallas as pl
from jax.experimental.pallas import tpu as pltpu
from jax.experimental.pallas import tpu_sc as plsc

N = 50000
E = 800000
D = 64
H = 128
EPS = 1e-7

NSC = 2
NTILES = 16
NPS = 25024
SROWS = 25088
STRIPE = SROWS // NTILES
TRASH = NPS
NPAD = NSC * NPS
EPT = E // NTILES
CHUNK = 80
NCHUNK = EPT // CHUNK
QC = D // 4
ACC = 2 * QC


def _sc_accumulate():
    mesh = plsc.VectorSubcoreMesh(core_axis_name="c", subcore_axis_name="s")

    @functools.partial(
        pl.kernel,
        out_type=tuple(
            jax.ShapeDtypeStruct((NPAD, ACC), jnp.float32) for _ in range(4)),
        mesh=mesh,
        scratch_types=[
            [pltpu.VMEM((CHUNK,), jnp.int32)] * 2,
            [pltpu.VMEM((CHUNK,), jnp.int32)] * 2,
            [pltpu.VMEM((CHUNK,), jnp.int32)] * 2,
            [pltpu.VMEM((CHUNK, QC), jnp.float32)] * 2,
            [pltpu.VMEM((CHUNK, QC), jnp.float32)] * 2,
            [pltpu.VMEM((CHUNK, ACC), jnp.float32)] * 2,
            pltpu.VMEM((STRIPE, ACC), jnp.float32),
            pltpu.VMEM_SHARED((SROWS, ACC), jnp.float32),
            [pltpu.SemaphoreType.DMA] * 2,
            [pltpu.SemaphoreType.DMA] * 2,
            [pltpu.SemaphoreType.DMA] * 2,
            [pltpu.SemaphoreType.DMA] * 2,
        ],
        compiler_params=pltpu.CompilerParams(use_tc_tiling_on_sc=False),
    )
    def k(x0_hbm, x1_hbm, x2_hbm, x3_hbm,
          esrc_hbm, edst_hbm,
          ea0_hbm, ea1_hbm, ea2_hbm, ea3_hbm,
          o0_hbm, o1_hbm, o2_hbm, o3_hbm,
          src_v, dst_v, sidx_v, xg_v, ea_v, vb_v, zb_v,
          acc_sh, sem_i, sem_g, sem_e, sem_s):
        c = lax.axis_index("c")
        s = lax.axis_index("s")
        base_node = c * NPS
        ebase = s * EPT
        xq = [x0_hbm, x1_hbm, x2_hbm, x3_hbm]
        eaq = [ea0_hbm, ea1_hbm, ea2_hbm, ea3_hbm]
        oq = [o0_hbm, o1_hbm, o2_hbm, o3_hbm]

        def issue_idx(j, b):
            base = ebase + j * CHUNK
            pltpu.async_copy(esrc_hbm.at[pl.ds(base, CHUNK)], src_v[b],
                             sem_i[b])
            pltpu.async_copy(edst_hbm.at[pl.ds(base, CHUNK)], dst_v[b],
                             sem_i[b])

        def wait_idx(b):
            pltpu.make_async_copy(esrc_hbm.at[pl.ds(0, CHUNK)], src_v[b],
                                  sem_i[b]).wait()
            pltpu.make_async_copy(edst_hbm.at[pl.ds(0, CHUNK)], dst_v[b],
                                  sem_i[b]).wait()

        def issue_ge(j, b, q):
            base = ebase + j * CHUNK
            pltpu.async_copy(xq[q].at[src_v[b]], xg_v[b], sem_g[b])
            pltpu.async_copy(eaq[q].at[pl.ds(base, CHUNK)], ea_v[b],
                             sem_e[b])

        def wait_ge(b, q):
            pltpu.make_async_copy(xq[q].at[src_v[b]], xg_v[b],
                                  sem_g[b]).wait()
            pltpu.make_async_copy(eaq[q].at[pl.ds(0, CHUNK)], ea_v[b],
                                  sem_e[b]).wait()

        def wait_scatter(b):
            pltpu.make_async_copy(vb_v[b], acc_sh.at[sidx_v[b]],
                                  sem_s[b]).wait()

        def chunk_body(j, b, q, wait_sc, do_idx=True, do_ge=True):
            del wait_sc
            wait_ge(b, q)
            for t in range(CHUNK // 16):
                dv = dst_v[b][pl.ds(t * 16, 16)]
                ld = dv - base_node
                ok = (ld >= 0) & (ld < NPS)
                sidx_v[b][pl.ds(t * 16, 16)] = jnp.where(ok, ld, TRASH)
            if do_idx is True:
                issue_idx(j + 2, b)
            elif do_idx is not None:
                @pl.when(do_idx)
                def _():
                    issue_idx(j + 2, b)
            if do_ge:
                wait_idx(b ^ 1)
                issue_ge(j + 1, b ^ 1, q)

            @pl.loop(0, CHUNK, unroll=8)
            def _(r):
                vx = xg_v[b][r, pl.ds(0, 16)]
                ve = ea_v[b][r, pl.ds(0, 16)]
                m = jnp.maximum(vx + ve, 0.0) + EPS
                p = jnp.exp(m)
                vb_v[b][r, pl.ds(0, 16)] = p
                vb_v[b][r, pl.ds(16, 16)] = m * p

            pltpu.sync_copy(vb_v[b], acc_sh.at[sidx_v[b]], add=True)

        @pl.loop(0, STRIPE)
        def _(i):
            z = jnp.zeros((16,), jnp.float32)
            zb_v[i, pl.ds(0, 16)] = z
            zb_v[i, pl.ds(16, 16)] = z

        npairs = NCHUNK // 2

        for q in range(4):
            pltpu.sync_copy(zb_v, acc_sh.at[pl.ds(s * STRIPE, STRIPE)])
            plsc.subcore_barrier()

            issue_idx(0, 0)
            wait_idx(0)
            issue_ge(0, 0, q)
            issue_idx(1, 1)

            @pl.loop(0, npairs)
            def _(i):
                j0 = 2 * i
                j1 = j0 + 1
                chunk_body(j0, 0, q, wait_sc=(j0 >= 2))
                chunk_body(j1, 1, q, wait_sc=(j1 >= 2),
                           do_idx=(j1 + 2 < NCHUNK))

            chunk_body(NCHUNK - 1, 0, q, wait_sc=True,
                       do_idx=None, do_ge=False)
            plsc.subcore_barrier()

            gbase = base_node + s * STRIPE

            @pl.when(s < NTILES - 1)
            def _():
                pltpu.sync_copy(acc_sh.at[pl.ds(s * STRIPE, STRIPE)],
                                oq[q].at[pl.ds(gbase, STRIPE)])

            last = NPS - (NTILES - 1) * STRIPE

            @pl.when(s == NTILES - 1)
            def _():
                pltpu.sync_copy(acc_sh.at[pl.ds(s * STRIPE, last)],
                                oq[q].at[pl.ds(gbase, last)])

            plsc.subcore_barrier()

    return k


_ROWS_BLK = 3128


def _mlp_body(a0_ref, a1_ref, a2_ref, a3_ref, x_ref, w1_ref, b1_ref,
              g_ref, bt_ref, w2_ref, b2_ref, o_ref):
    quarters = [a0_ref[...], a1_ref[...], a2_ref[...], a3_ref[...]]
    s_acc = jnp.concatenate([a[:, :QC] for a in quarters], axis=1)
    w_acc = jnp.concatenate([a[:, QC:] for a in quarters], axis=1)
    agg = w_acc / (s_acc + 1e-16)
    out = agg + x_ref[...]
    hpre = jnp.dot(out, w1_ref[...],
                   preferred_element_type=jnp.float32) + b1_ref[...]
    mu = jnp.mean(hpre, axis=-1, keepdims=True)
    var = jnp.mean((hpre - mu) ** 2, axis=-1, keepdims=True)
    hn = (hpre - mu) * lax.rsqrt(var + 1e-5) * g_ref[...] + bt_ref[...]
    hr = jnp.maximum(hn, 0.0)
    o_ref[...] = jnp.dot(hr, w2_ref[...],
                         preferred_element_type=jnp.float32) + b2_ref[...]


def _mlp(accs, x, W1, b1, ln_g, ln_b, W2, b2):
    nblk = pl.cdiv(N, _ROWS_BLK)
    return pl.pallas_call(
        _mlp_body,
        out_shape=jax.ShapeDtypeStruct((N, D), jnp.float32),
        grid=(nblk,),
        in_specs=[
            pl.BlockSpec((_ROWS_BLK, ACC), lambda i: (i, 0)),
            pl.BlockSpec((_ROWS_BLK, ACC), lambda i: (i, 0)),
            pl.BlockSpec((_ROWS_BLK, ACC), lambda i: (i, 0)),
            pl.BlockSpec((_ROWS_BLK, ACC), lambda i: (i, 0)),
            pl.BlockSpec((_ROWS_BLK, D), lambda i: (i, 0)),
            pl.BlockSpec((D, H), lambda i: (0, 0)),
            pl.BlockSpec((1, H), lambda i: (0, 0)),
            pl.BlockSpec((1, H), lambda i: (0, 0)),
            pl.BlockSpec((1, H), lambda i: (0, 0)),
            pl.BlockSpec((H, D), lambda i: (0, 0)),
            pl.BlockSpec((1, D), lambda i: (0, 0)),
        ],
        out_specs=pl.BlockSpec((_ROWS_BLK, D), lambda i: (i, 0)),
    )(*accs, x, W1, b1, ln_g, ln_b, W2, b2)


def kernel(x, edge_index, edge_attr, W1, b1, ln_g, ln_b, W2, b2):
    xqs = [x[:, i * QC:(i + 1) * QC] for i in range(4)]
    eaqs = [edge_attr[:, i * QC:(i + 1) * QC] for i in range(4)]
    accs = _sc_accumulate()(*xqs, edge_index[0], edge_index[1], *eaqs)
    accs = [a[:N] for a in accs]
    return _mlp(accs, x,
                W1, b1.reshape(1, H), ln_g.reshape(1, H),
                ln_b.reshape(1, H), W2, b2.reshape(1, D))


# speedup vs baseline: 2.0277x; 1.0697x over previous
import functools

import jax
import jax.numpy as jnp
from jax import lax
from jax.experimental import pallas as pl
from jax.experimental.pallas import tpu as pltpu
from jax.experimental.pallas import tpu_sc as plsc

N = 50000
E = 800000
D = 64
H = 128
EPS = 1e-7

NSC = 2
NTILES = 16
NPS = 25024
SROWS = 25088
STRIPE = SROWS // NTILES
TRASH = NPS
NPAD = NSC * NPS
EPT = E // NTILES
CHUNK = 80
NCHUNK = EPT // CHUNK
QC = D // 4
ACC = 2 * QC


def _sc_accumulate():
    mesh = plsc.VectorSubcoreMesh(core_axis_name="c", subcore_axis_name="s")

    @functools.partial(
        pl.kernel,
        out_type=tuple(
            jax.ShapeDtypeStruct((NPAD, ACC), jnp.float32) for _ in range(4)),
        mesh=mesh,
        scratch_types=[
            [pltpu.VMEM((CHUNK,), jnp.int32)] * 2,
            [pltpu.VMEM((CHUNK,), jnp.int32)] * 2,
            [pltpu.VMEM((CHUNK,), jnp.int32)] * 2,
            [pltpu.VMEM((CHUNK, QC), jnp.float32)] * 2,
            [pltpu.VMEM((CHUNK, QC), jnp.float32)] * 2,
            [pltpu.VMEM((CHUNK, ACC), jnp.float32)] * 2,
            pltpu.VMEM((STRIPE, ACC), jnp.float32),
            pltpu.VMEM_SHARED((SROWS, ACC), jnp.float32),
            [pltpu.SemaphoreType.DMA] * 2,
            [pltpu.SemaphoreType.DMA] * 2,
            [pltpu.SemaphoreType.DMA] * 2,
            [pltpu.SemaphoreType.DMA] * 2,
        ],
        compiler_params=pltpu.CompilerParams(use_tc_tiling_on_sc=False),
    )
    def k(x0_hbm, x1_hbm, x2_hbm, x3_hbm,
          esrc_hbm, edst_hbm,
          ea0_hbm, ea1_hbm, ea2_hbm, ea3_hbm,
          o0_hbm, o1_hbm, o2_hbm, o3_hbm,
          src_v, dst_v, sidx_v, xg_v, ea_v, vb_v, zb_v,
          acc_sh, sem_i, sem_g, sem_e, sem_s):
        c = lax.axis_index("c")
        s = lax.axis_index("s")
        base_node = c * NPS
        ebase = s * EPT
        xq = [x0_hbm, x1_hbm, x2_hbm, x3_hbm]
        eaq = [ea0_hbm, ea1_hbm, ea2_hbm, ea3_hbm]
        oq = [o0_hbm, o1_hbm, o2_hbm, o3_hbm]

        def issue_idx(j, b):
            base = ebase + j * CHUNK
            pltpu.async_copy(esrc_hbm.at[pl.ds(base, CHUNK)], src_v[b],
                             sem_i[b])
            pltpu.async_copy(edst_hbm.at[pl.ds(base, CHUNK)], dst_v[b],
                             sem_i[b])

        def wait_idx(b):
            pltpu.make_async_copy(esrc_hbm.at[pl.ds(0, CHUNK)], src_v[b],
                                  sem_i[b]).wait()
            pltpu.make_async_copy(edst_hbm.at[pl.ds(0, CHUNK)], dst_v[b],
                                  sem_i[b]).wait()

        def issue_ge(j, b, q):
            base = ebase + j * CHUNK
            pltpu.async_copy(xq[q].at[src_v[b]], xg_v[b], sem_g[b])
            pltpu.async_copy(eaq[q].at[pl.ds(base, CHUNK)], ea_v[b],
                             sem_e[b])

        def wait_ge(b, q):
            pltpu.make_async_copy(xq[q].at[src_v[b]], xg_v[b],
                                  sem_g[b]).wait()
            pltpu.make_async_copy(eaq[q].at[pl.ds(0, CHUNK)], ea_v[b],
                                  sem_e[b]).wait()

        def wait_scatter(b):
            pltpu.make_async_copy(vb_v[b], acc_sh.at[sidx_v[b]],
                                  sem_s[b]).wait()

        def chunk_body(j, b, q, wait_sc, do_idx=True, do_ge=True):
            wait_ge(b, q)
            for t in range(CHUNK // 16):
                dv = dst_v[b][pl.ds(t * 16, 16)]
                ld = dv - base_node
                ok = (ld >= 0) & (ld < NPS)
                sidx_v[b][pl.ds(t * 16, 16)] = jnp.where(ok, ld, TRASH)
            if do_idx is True:
                issue_idx(j + 2, b)
            elif do_idx is not None:
                @pl.when(do_idx)
                def _():
                    issue_idx(j + 2, b)
            if do_ge:
                wait_idx(b ^ 1)
                issue_ge(j + 1, b ^ 1, q)

            @pl.loop(0, CHUNK, unroll=8)
            def _(r):
                vx = xg_v[b][r, pl.ds(0, 16)]
                ve = ea_v[b][r, pl.ds(0, 16)]
                m = jnp.maximum(vx + ve, 0.0) + EPS
                p = jnp.exp(m)
                vb_v[b][r, pl.ds(0, 16)] = p
                vb_v[b][r, pl.ds(16, 16)] = m * p

            if wait_sc is True:
                wait_scatter(b ^ 1)
            elif wait_sc is not None:
                @pl.when(wait_sc)
                def _():
                    wait_scatter(b ^ 1)
            pltpu.async_copy(vb_v[b], acc_sh.at[sidx_v[b]], sem_s[b],
                             add=True)

        @pl.loop(0, STRIPE)
        def _(i):
            z = jnp.zeros((16,), jnp.float32)
            zb_v[i, pl.ds(0, 16)] = z
            zb_v[i, pl.ds(16, 16)] = z

        npairs = NCHUNK // 2

        for q in range(4):
            pltpu.sync_copy(zb_v, acc_sh.at[pl.ds(s * STRIPE, STRIPE)])
            plsc.subcore_barrier()

            issue_idx(0, 0)
            wait_idx(0)
            issue_ge(0, 0, q)
            issue_idx(1, 1)

            @pl.loop(0, npairs)
            def _(i):
                j0 = 2 * i
                j1 = j0 + 1
                chunk_body(j0, 0, q, wait_sc=(j0 >= 1))
                chunk_body(j1, 1, q, wait_sc=True,
                           do_idx=(j1 + 2 < NCHUNK))

            chunk_body(NCHUNK - 1, 0, q, wait_sc=True,
                       do_idx=None, do_ge=False)
            wait_scatter(0)
            plsc.subcore_barrier()

            gbase = base_node + s * STRIPE

            @pl.when(s < NTILES - 1)
            def _():
                pltpu.sync_copy(acc_sh.at[pl.ds(s * STRIPE, STRIPE)],
                                oq[q].at[pl.ds(gbase, STRIPE)])

            last = NPS - (NTILES - 1) * STRIPE

            @pl.when(s == NTILES - 1)
            def _():
                pltpu.sync_copy(acc_sh.at[pl.ds(s * STRIPE, last)],
                                oq[q].at[pl.ds(gbase, last)])

            plsc.subcore_barrier()

    return k


_ROWS_BLK = 3128


def _mlp_body(a0_ref, a1_ref, a2_ref, a3_ref, x_ref, w1_ref, b1_ref,
              g_ref, bt_ref, w2_ref, b2_ref, o_ref):
    quarters = [a0_ref[...], a1_ref[...], a2_ref[...], a3_ref[...]]
    s_acc = jnp.concatenate([a[:, :QC] for a in quarters], axis=1)
    w_acc = jnp.concatenate([a[:, QC:] for a in quarters], axis=1)
    agg = w_acc / (s_acc + 1e-16)
    out = agg + x_ref[...]
    hpre = jnp.dot(out, w1_ref[...],
                   preferred_element_type=jnp.float32) + b1_ref[...]
    mu = jnp.mean(hpre, axis=-1, keepdims=True)
    var = jnp.mean((hpre - mu) ** 2, axis=-1, keepdims=True)
    hn = (hpre - mu) * lax.rsqrt(var + 1e-5) * g_ref[...] + bt_ref[...]
    hr = jnp.maximum(hn, 0.0)
    o_ref[...] = jnp.dot(hr, w2_ref[...],
                         preferred_element_type=jnp.float32) + b2_ref[...]


def _mlp(accs, x, W1, b1, ln_g, ln_b, W2, b2):
    nblk = pl.cdiv(N, _ROWS_BLK)
    return pl.pallas_call(
        _mlp_body,
        out_shape=jax.ShapeDtypeStruct((N, D), jnp.float32),
        grid=(nblk,),
        in_specs=[
            pl.BlockSpec((_ROWS_BLK, ACC), lambda i: (i, 0)),
            pl.BlockSpec((_ROWS_BLK, ACC), lambda i: (i, 0)),
            pl.BlockSpec((_ROWS_BLK, ACC), lambda i: (i, 0)),
            pl.BlockSpec((_ROWS_BLK, ACC), lambda i: (i, 0)),
            pl.BlockSpec((_ROWS_BLK, D), lambda i: (i, 0)),
            pl.BlockSpec((D, H), lambda i: (0, 0)),
            pl.BlockSpec((1, H), lambda i: (0, 0)),
            pl.BlockSpec((1, H), lambda i: (0, 0)),
            pl.BlockSpec((1, H), lambda i: (0, 0)),
            pl.BlockSpec((H, D), lambda i: (0, 0)),
            pl.BlockSpec((1, D), lambda i: (0, 0)),
        ],
        out_specs=pl.BlockSpec((_ROWS_BLK, D), lambda i: (i, 0)),
    )(*accs, x, W1, b1, ln_g, ln_b, W2, b2)


def kernel(x, edge_index, edge_attr, W1, b1, ln_g, ln_b, W2, b2):
    xqs = [x[:, i * QC:(i + 1) * QC] for i in range(4)]
    eaqs = [edge_attr[:, i * QC:(i + 1) * QC] for i in range(4)]
    accs = _sc_accumulate()(*xqs, edge_index[0], edge_index[1], *eaqs)
    accs = [a[:N] for a in accs]
    return _mlp(accs, x,
                W1, b1.reshape(1, H), ln_g.reshape(1, H),
                ln_b.reshape(1, H), W2, b2.reshape(1, D))


# speedup vs baseline: 2.7854x; 1.3737x over previous
import functools

import jax
import jax.numpy as jnp
from jax import lax
from jax.experimental import pallas as pl
from jax.experimental.pallas import tpu as pltpu
from jax.experimental.pallas import tpu_sc as plsc

N = 50000
E = 800000
D = 64
H = 128
EPS = 1e-7

NSC = 2
NTILES = 16
NPS = 25024
SROWS = 25088
STRIPE = SROWS // NTILES
TRASH = NPS
NPAD = NSC * NPS
EPT = E // NTILES
CHUNK = 80
NCHUNK = EPT // CHUNK
QC = D // 4
ACC = 2 * QC


def _sc_accumulate():
    mesh = plsc.VectorSubcoreMesh(core_axis_name="c", subcore_axis_name="s")

    @functools.partial(
        pl.kernel,
        out_type=tuple(
            jax.ShapeDtypeStruct((NPAD, ACC), jnp.float32) for _ in range(4)),
        mesh=mesh,
        scratch_types=[
            [pltpu.VMEM((CHUNK,), jnp.int32)] * 4,
            [pltpu.VMEM((CHUNK,), jnp.int32)] * 4,
            [pltpu.VMEM((CHUNK,), jnp.int32)] * 4,
            [pltpu.VMEM((CHUNK, QC), jnp.float32)] * 4,
            [pltpu.VMEM((CHUNK, QC), jnp.float32)] * 4,
            [pltpu.VMEM((CHUNK, ACC), jnp.float32)] * 4,
            pltpu.VMEM((STRIPE, ACC), jnp.float32),
            pltpu.VMEM_SHARED((SROWS, ACC), jnp.float32),
            [pltpu.SemaphoreType.DMA] * 4,
            [pltpu.SemaphoreType.DMA] * 4,
            [pltpu.SemaphoreType.DMA] * 4,
            [pltpu.SemaphoreType.DMA] * 4,
        ],
        compiler_params=pltpu.CompilerParams(use_tc_tiling_on_sc=False),
    )
    def k(x0_hbm, x1_hbm, x2_hbm, x3_hbm,
          esrc_hbm, edst_hbm, ea_hbm,
          o0_hbm, o1_hbm, o2_hbm, o3_hbm,
          src_v, dst_v, sidx_v, xg_v, ea_v, vb_v, zb_v,
          acc_sh, sem_i, sem_g, sem_e, sem_s):
        c = lax.axis_index("c")
        s = lax.axis_index("s")
        base_node = c * NPS
        ebase = s * EPT
        xq = [x0_hbm, x1_hbm, x2_hbm, x3_hbm]
        oq = [o0_hbm, o1_hbm, o2_hbm, o3_hbm]

        def issue_idx(j, b):
            base = ebase + j * CHUNK
            pltpu.async_copy(esrc_hbm.at[pl.ds(base, CHUNK)], src_v[b],
                             sem_i[b])
            pltpu.async_copy(edst_hbm.at[pl.ds(base, CHUNK)], dst_v[b],
                             sem_i[b])

        def wait_idx(b):
            pltpu.make_async_copy(esrc_hbm.at[pl.ds(0, CHUNK)], src_v[b],
                                  sem_i[b]).wait()
            pltpu.make_async_copy(edst_hbm.at[pl.ds(0, CHUNK)], dst_v[b],
                                  sem_i[b]).wait()

        def issue_ge(j, b, q):
            base = ebase + j * CHUNK
            pltpu.async_copy(xq[q].at[src_v[b]], xg_v[b], sem_g[b])
            pltpu.async_copy(
                ea_hbm.at[pl.ds(base, CHUNK), pl.ds(q * QC, QC)],
                ea_v[b], sem_e[b])

        def wait_ge(b, q):
            pltpu.make_async_copy(xq[q].at[src_v[b]], xg_v[b],
                                  sem_g[b]).wait()
            pltpu.make_async_copy(
                ea_hbm.at[pl.ds(0, CHUNK), pl.ds(q * QC, QC)],
                ea_v[b], sem_e[b]).wait()

        def wait_scatter(b):
            pltpu.make_async_copy(vb_v[b], acc_sh.at[sidx_v[b]],
                                  sem_s[b]).wait()

        def chunk_body(j, b, q, wait_sc, do_idx=True, do_ge=True):
            wait_ge(b, q)
            for t in range(CHUNK // 16):
                dv = dst_v[b][pl.ds(t * 16, 16)]
                ld = dv - base_node
                ok = (ld >= 0) & (ld < NPS)
                sidx_v[b][pl.ds(t * 16, 16)] = jnp.where(ok, ld, TRASH)
            if do_idx is True:
                issue_idx(j + 4, b)
            elif do_idx is not None:
                @pl.when(do_idx)
                def _():
                    issue_idx(j + 4, b)
            if do_ge is True:
                wait_idx((b + 3) & 3)
                issue_ge(j + 3, (b + 3) & 3, q)
            elif do_ge is not None:
                @pl.when(do_ge)
                def _():
                    wait_idx((b + 3) & 3)
                    issue_ge(j + 3, (b + 3) & 3, q)

            @pl.loop(0, CHUNK, unroll=8)
            def _(r):
                vx = xg_v[b][r, pl.ds(0, 16)]
                ve = ea_v[b][r, pl.ds(0, 16)]
                m = jnp.maximum(vx + ve, 0.0) + EPS
                p = jnp.exp(m)
                vb_v[b][r, pl.ds(0, 16)] = p
                vb_v[b][r, pl.ds(16, 16)] = m * p

            if wait_sc is True:
                wait_scatter((b + 3) & 3)
            elif wait_sc is not None:
                @pl.when(wait_sc)
                def _():
                    wait_scatter((b + 3) & 3)
            pltpu.async_copy(vb_v[b], acc_sh.at[sidx_v[b]], sem_s[b],
                             add=True)

        @pl.loop(0, STRIPE)
        def _(i):
            z = jnp.zeros((16,), jnp.float32)
            zb_v[i, pl.ds(0, 16)] = z
            zb_v[i, pl.ds(16, 16)] = z

        nquads = NCHUNK // 4

        for q in range(4):
            pltpu.sync_copy(zb_v, acc_sh.at[pl.ds(s * STRIPE, STRIPE)])
            plsc.subcore_barrier()

            for b in range(3):
                issue_idx(b, b)
            for b in range(3):
                wait_idx(b)
                issue_ge(b, b, q)
            issue_idx(3, 3)

            @pl.loop(0, nquads)
            def _(i):
                for db in range(4):
                    j = 4 * i + db
                    chunk_body(j, db, q,
                               wait_sc=(j >= 1),
                               do_idx=(j + 4 < NCHUNK),
                               do_ge=(j + 3 < NCHUNK))

            chunk_body(NCHUNK - 1, (NCHUNK - 1) & 3, q, wait_sc=True,
                       do_idx=None, do_ge=None)
            wait_scatter((NCHUNK - 1) & 3)
            plsc.subcore_barrier()

            gbase = base_node + s * STRIPE

            @pl.when(s < NTILES - 1)
            def _():
                pltpu.sync_copy(acc_sh.at[pl.ds(s * STRIPE, STRIPE)],
                                oq[q].at[pl.ds(gbase, STRIPE)])

            last = NPS - (NTILES - 1) * STRIPE

            @pl.when(s == NTILES - 1)
            def _():
                pltpu.sync_copy(acc_sh.at[pl.ds(s * STRIPE, last)],
                                oq[q].at[pl.ds(gbase, last)])

            plsc.subcore_barrier()

    return k


_ROWS_BLK = 3128


def _mlp_body(a0_ref, a1_ref, a2_ref, a3_ref, x_ref, w1_ref, b1_ref,
              g_ref, bt_ref, w2_ref, b2_ref, o_ref):
    quarters = [a0_ref[...], a1_ref[...], a2_ref[...], a3_ref[...]]
    s_acc = jnp.concatenate([a[:, :QC] for a in quarters], axis=1)
    w_acc = jnp.concatenate([a[:, QC:] for a in quarters], axis=1)
    agg = w_acc / (s_acc + 1e-16)
    out = agg + x_ref[...]
    hpre = jnp.dot(out, w1_ref[...],
                   preferred_element_type=jnp.float32) + b1_ref[...]
    mu = jnp.mean(hpre, axis=-1, keepdims=True)
    var = jnp.mean((hpre - mu) ** 2, axis=-1, keepdims=True)
    hn = (hpre - mu) * lax.rsqrt(var + 1e-5) * g_ref[...] + bt_ref[...]
    hr = jnp.maximum(hn, 0.0)
    o_ref[...] = jnp.dot(hr, w2_ref[...],
                         preferred_element_type=jnp.float32) + b2_ref[...]


def _mlp(accs, x, W1, b1, ln_g, ln_b, W2, b2):
    nblk = pl.cdiv(N, _ROWS_BLK)
    return pl.pallas_call(
        _mlp_body,
        out_shape=jax.ShapeDtypeStruct((N, D), jnp.float32),
        grid=(nblk,),
        in_specs=[
            pl.BlockSpec((_ROWS_BLK, ACC), lambda i: (i, 0)),
            pl.BlockSpec((_ROWS_BLK, ACC), lambda i: (i, 0)),
            pl.BlockSpec((_ROWS_BLK, ACC), lambda i: (i, 0)),
            pl.BlockSpec((_ROWS_BLK, ACC), lambda i: (i, 0)),
            pl.BlockSpec((_ROWS_BLK, D), lambda i: (i, 0)),
            pl.BlockSpec((D, H), lambda i: (0, 0)),
            pl.BlockSpec((1, H), lambda i: (0, 0)),
            pl.BlockSpec((1, H), lambda i: (0, 0)),
            pl.BlockSpec((1, H), lambda i: (0, 0)),
            pl.BlockSpec((H, D), lambda i: (0, 0)),
            pl.BlockSpec((1, D), lambda i: (0, 0)),
        ],
        out_specs=pl.BlockSpec((_ROWS_BLK, D), lambda i: (i, 0)),
    )(*accs, x, W1, b1, ln_g, ln_b, W2, b2)


def kernel(x, edge_index, edge_attr, W1, b1, ln_g, ln_b, W2, b2):
    xqs = [x[:, i * QC:(i + 1) * QC] for i in range(4)]
    accs = _sc_accumulate()(*xqs, edge_index[0], edge_index[1], edge_attr)
    accs = [a[:N] for a in accs]
    return _mlp(accs, x,
                W1, b1.reshape(1, H), ln_g.reshape(1, H),
                ln_b.reshape(1, H), W2, b2.reshape(1, D))


# speedup vs baseline: 2.8802x; 1.0340x over previous
import functools

import jax
import jax.numpy as jnp
from jax import lax
from jax.experimental import pallas as pl
from jax.experimental.pallas import tpu as pltpu
from jax.experimental.pallas import tpu_sc as plsc

N = 50000
E = 800000
D = 64
H = 128
EPS = 1e-7

NSC = 2
NTILES = 16
NPS = 25024
SROWS = 25088
STRIPE = SROWS // NTILES
TRASH = NPS
NPAD = NSC * NPS
EPT = E // NTILES
CHUNK = 128
NCHUNK = EPT // CHUNK
TAIL = EPT - NCHUNK * CHUNK
QC = D // 4
ACC = 2 * QC


XROWS = 250
XSPAN = 2000
XWORK = N // XSPAN


def _sc_quarter_x():
    mesh = plsc.VectorSubcoreMesh(core_axis_name="c", subcore_axis_name="s")

    @functools.partial(
        pl.kernel,
        out_type=tuple(
            jax.ShapeDtypeStruct((N, QC), jnp.float32) for _ in range(4)),
        mesh=mesh,
        scratch_types=[
            pltpu.VMEM((XROWS, D), jnp.float32),
        ],
        compiler_params=pltpu.CompilerParams(use_tc_tiling_on_sc=False),
    )
    def k(x_hbm, q0_hbm, q1_hbm, q2_hbm, q3_hbm, buf_v):
        c = lax.axis_index("c")
        s = lax.axis_index("s")
        w = s * NSC + c
        base = w * XSPAN
        oq = [q0_hbm, q1_hbm, q2_hbm, q3_hbm]

        @pl.when(w < XWORK)
        def _():
            @pl.loop(0, XSPAN // XROWS)
            def _(i):
                row0 = base + i * XROWS
                pltpu.sync_copy(x_hbm.at[pl.ds(row0, XROWS)], buf_v)
                for q in range(4):
                    pltpu.sync_copy(
                        buf_v.at[pl.ds(0, XROWS), pl.ds(q * QC, QC)],
                        oq[q].at[pl.ds(row0, XROWS)])

    return k


def _sc_accumulate():
    mesh = plsc.VectorSubcoreMesh(core_axis_name="c", subcore_axis_name="s")

    @functools.partial(
        pl.kernel,
        out_type=tuple(
            jax.ShapeDtypeStruct((NPAD, ACC), jnp.float32) for _ in range(4)),
        mesh=mesh,
        scratch_types=[
            [pltpu.VMEM((CHUNK,), jnp.int32)] * 4,
            [pltpu.VMEM((CHUNK,), jnp.int32)] * 4,
            [pltpu.VMEM((CHUNK,), jnp.int32)] * 4,
            [pltpu.VMEM((CHUNK, QC), jnp.float32)] * 4,
            [pltpu.VMEM((CHUNK, QC), jnp.float32)] * 4,
            [pltpu.VMEM((CHUNK, ACC), jnp.float32)] * 2,
            pltpu.VMEM((STRIPE, ACC), jnp.float32),
            pltpu.VMEM_SHARED((SROWS, ACC), jnp.float32),
            [pltpu.SemaphoreType.DMA] * 4,
            [pltpu.SemaphoreType.DMA] * 4,
            [pltpu.SemaphoreType.DMA] * 4,
            [pltpu.SemaphoreType.DMA] * 2,
        ],
        compiler_params=pltpu.CompilerParams(use_tc_tiling_on_sc=False),
    )
    def k(x0_hbm, x1_hbm, x2_hbm, x3_hbm,
          esrc_hbm, edst_hbm, ea_hbm,
          o0_hbm, o1_hbm, o2_hbm, o3_hbm,
          src_v, dst_v, sidx_v, xg_v, ea_v, vb_v, zb_v,
          acc_sh, sem_i, sem_g, sem_e, sem_s):
        c = lax.axis_index("c")
        s = lax.axis_index("s")
        base_node = c * NPS
        ebase = s * EPT
        xq = [x0_hbm, x1_hbm, x2_hbm, x3_hbm]
        oq = [o0_hbm, o1_hbm, o2_hbm, o3_hbm]

        def issue_idx(j, b):
            base = ebase + j * CHUNK
            pltpu.async_copy(esrc_hbm.at[pl.ds(base, CHUNK)], src_v[b],
                             sem_i[b])
            pltpu.async_copy(edst_hbm.at[pl.ds(base, CHUNK)], dst_v[b],
                             sem_i[b])

        def wait_idx(b):
            pltpu.make_async_copy(esrc_hbm.at[pl.ds(0, CHUNK)], src_v[b],
                                  sem_i[b]).wait()
            pltpu.make_async_copy(edst_hbm.at[pl.ds(0, CHUNK)], dst_v[b],
                                  sem_i[b]).wait()

        def issue_ge(j, b, q):
            base = ebase + j * CHUNK
            pltpu.async_copy(xq[q].at[src_v[b]], xg_v[b], sem_g[b])
            pltpu.async_copy(
                ea_hbm.at[pl.ds(base, CHUNK), pl.ds(q * QC, QC)],
                ea_v[b], sem_e[b])

        def wait_ge(b, q):
            pltpu.make_async_copy(xq[q].at[src_v[b]], xg_v[b],
                                  sem_g[b]).wait()
            pltpu.make_async_copy(
                ea_hbm.at[pl.ds(0, CHUNK), pl.ds(q * QC, QC)],
                ea_v[b], sem_e[b]).wait()

        def wait_scatter(b2, b4):
            pltpu.make_async_copy(vb_v[b2], acc_sh.at[sidx_v[b4]],
                                  sem_s[b2]).wait()

        def chunk_body(j, b, q, wait_sc, do_idx=True, do_ge=True):
            wait_ge(b, q)
            for t in range(CHUNK // 16):
                dv = dst_v[b][pl.ds(t * 16, 16)]
                ld = dv - base_node
                ok = (ld >= 0) & (ld < NPS)
                sidx_v[b][pl.ds(t * 16, 16)] = jnp.where(ok, ld, TRASH)
            if do_idx is True:
                issue_idx(j + 4, b)
            elif do_idx is not None:
                @pl.when(do_idx)
                def _():
                    issue_idx(j + 4, b)
            if do_ge is True:
                wait_idx((b + 3) & 3)
                issue_ge(j + 3, (b + 3) & 3, q)
            elif do_ge is not None:
                @pl.when(do_ge)
                def _():
                    wait_idx((b + 3) & 3)
                    issue_ge(j + 3, (b + 3) & 3, q)

            b2 = b & 1

            @pl.loop(0, CHUNK, unroll=8)
            def _(r):
                vx = xg_v[b][r, pl.ds(0, 16)]
                ve = ea_v[b][r, pl.ds(0, 16)]
                m = jnp.maximum(vx + ve, 0.0) + EPS
                p = jnp.exp(m)
                vb_v[b2][r, pl.ds(0, 16)] = p
                vb_v[b2][r, pl.ds(16, 16)] = m * p

            if wait_sc is True:
                wait_scatter(b2 ^ 1, (b + 3) & 3)
            elif wait_sc is not None:
                @pl.when(wait_sc)
                def _():
                    wait_scatter(b2 ^ 1, (b + 3) & 3)
            pltpu.async_copy(vb_v[b2], acc_sh.at[sidx_v[b]], sem_s[b2],
                             add=True)

        @pl.loop(0, STRIPE)
        def _(i):
            z = jnp.zeros((16,), jnp.float32)
            zb_v[i, pl.ds(0, 16)] = z
            zb_v[i, pl.ds(16, 16)] = z

        nquads = NCHUNK // 4

        for q in range(4):
            pltpu.sync_copy(zb_v, acc_sh.at[pl.ds(s * STRIPE, STRIPE)])
            plsc.subcore_barrier()

            for b in range(3):
                issue_idx(b, b)
            for b in range(3):
                wait_idx(b)
                issue_ge(b, b, q)
            issue_idx(3, 3)

            @pl.loop(0, nquads)
            def _(i):
                for db in range(4):
                    j = 4 * i + db
                    chunk_body(j, db, q,
                               wait_sc=(j >= 1),
                               do_idx=(j + 4 < NCHUNK),
                               do_ge=(j + 3 < NCHUNK))

            chunk_body(NCHUNK - 2, (NCHUNK - 2) & 3, q, wait_sc=True,
                       do_idx=None, do_ge=None)
            chunk_body(NCHUNK - 1, (NCHUNK - 1) & 3, q, wait_sc=True,
                       do_idx=None, do_ge=None)

            tb = NCHUNK & 3
            tbase = ebase + NCHUNK * CHUNK
            pltpu.sync_copy(esrc_hbm.at[pl.ds(tbase, TAIL)],
                            src_v[tb].at[pl.ds(0, TAIL)])
            pltpu.sync_copy(edst_hbm.at[pl.ds(tbase, TAIL)],
                            dst_v[tb].at[pl.ds(0, TAIL)])
            pltpu.async_copy(xq[q].at[src_v[tb]], xg_v[tb], sem_g[tb])
            pltpu.sync_copy(
                ea_hbm.at[pl.ds(tbase, TAIL), pl.ds(q * QC, QC)],
                ea_v[tb].at[pl.ds(0, TAIL)])
            for t in range(TAIL // 16):
                dv = dst_v[tb][pl.ds(t * 16, 16)]
                ld = dv - base_node
                ok = (ld >= 0) & (ld < NPS)
                sidx_v[tb][pl.ds(t * 16, 16)] = jnp.where(ok, ld, TRASH)
            for t in range(TAIL // 16, CHUNK // 16):
                sidx_v[tb][pl.ds(t * 16, 16)] = jnp.full(
                    (16,), TRASH, jnp.int32)
            pltpu.make_async_copy(xq[q].at[src_v[tb]], xg_v[tb],
                                  sem_g[tb]).wait()

            tb2 = tb & 1

            @pl.loop(0, CHUNK, unroll=8)
            def _(r):
                vx = xg_v[tb][r, pl.ds(0, 16)]
                ve = ea_v[tb][r, pl.ds(0, 16)]
                m = jnp.maximum(vx + ve, 0.0) + EPS
                p = jnp.exp(m)
                vb_v[tb2][r, pl.ds(0, 16)] = p
                vb_v[tb2][r, pl.ds(16, 16)] = m * p

            wait_scatter((NCHUNK - 1) & 1, (NCHUNK - 1) & 3)
            pltpu.sync_copy(vb_v[tb2], acc_sh.at[sidx_v[tb]], add=True)
            plsc.subcore_barrier()

            gbase = base_node + s * STRIPE

            @pl.when(s < NTILES - 1)
            def _():
                pltpu.sync_copy(acc_sh.at[pl.ds(s * STRIPE, STRIPE)],
                                oq[q].at[pl.ds(gbase, STRIPE)])

            last = NPS - (NTILES - 1) * STRIPE

            @pl.when(s == NTILES - 1)
            def _():
                pltpu.sync_copy(acc_sh.at[pl.ds(s * STRIPE, last)],
                                oq[q].at[pl.ds(gbase, last)])

            plsc.subcore_barrier()

    return k


_ROWS_BLK = 3128


def _mlp_body(a0_ref, a1_ref, a2_ref, a3_ref, x_ref, w1_ref, b1_ref,
              g_ref, bt_ref, w2_ref, b2_ref, o_ref):
    quarters = [a0_ref[...], a1_ref[...], a2_ref[...], a3_ref[...]]
    s_acc = jnp.concatenate([a[:, :QC] for a in quarters], axis=1)
    w_acc = jnp.concatenate([a[:, QC:] for a in quarters], axis=1)
    agg = w_acc / (s_acc + 1e-16)
    out = agg + x_ref[...]
    hpre = jnp.dot(out, w1_ref[...],
                   preferred_element_type=jnp.float32) + b1_ref[...]
    mu = jnp.mean(hpre, axis=-1, keepdims=True)
    var = jnp.mean((hpre - mu) ** 2, axis=-1, keepdims=True)
    hn = (hpre - mu) * lax.rsqrt(var + 1e-5) * g_ref[...] + bt_ref[...]
    hr = jnp.maximum(hn, 0.0)
    o_ref[...] = jnp.dot(hr, w2_ref[...],
                         preferred_element_type=jnp.float32) + b2_ref[...]


def _mlp(accs, x, W1, b1, ln_g, ln_b, W2, b2):
    nblk = pl.cdiv(N, _ROWS_BLK)
    return pl.pallas_call(
        _mlp_body,
        out_shape=jax.ShapeDtypeStruct((N, D), jnp.float32),
        grid=(nblk,),
        in_specs=[
            pl.BlockSpec((_ROWS_BLK, ACC), lambda i: (i, 0)),
            pl.BlockSpec((_ROWS_BLK, ACC), lambda i: (i, 0)),
            pl.BlockSpec((_ROWS_BLK, ACC), lambda i: (i, 0)),
            pl.BlockSpec((_ROWS_BLK, ACC), lambda i: (i, 0)),
            pl.BlockSpec((_ROWS_BLK, D), lambda i: (i, 0)),
            pl.BlockSpec((D, H), lambda i: (0, 0)),
            pl.BlockSpec((1, H), lambda i: (0, 0)),
            pl.BlockSpec((1, H), lambda i: (0, 0)),
            pl.BlockSpec((1, H), lambda i: (0, 0)),
            pl.BlockSpec((H, D), lambda i: (0, 0)),
            pl.BlockSpec((1, D), lambda i: (0, 0)),
        ],
        out_specs=pl.BlockSpec((_ROWS_BLK, D), lambda i: (i, 0)),
    )(*accs, x, W1, b1, ln_g, ln_b, W2, b2)


def kernel(x, edge_index, edge_attr, W1, b1, ln_g, ln_b, W2, b2):
    xqs = _sc_quarter_x()(x)
    accs = _sc_accumulate()(*xqs, edge_index[0], edge_index[1], edge_attr)
    accs = [a[:N] for a in accs]
    return _mlp(accs, x,
                W1, b1.reshape(1, H), ln_g.reshape(1, H),
                ln_b.reshape(1, H), W2, b2.reshape(1, D))


# speedup vs baseline: 4.0912x; 1.4204x over previous
import functools

import jax
import jax.numpy as jnp
from jax import lax
from jax.experimental import pallas as pl
from jax.experimental.pallas import tpu as pltpu
from jax.experimental.pallas import tpu_sc as plsc

N = 50000
E = 800000
D = 64
H = 128
EPS = 1e-7

NSC = 2
NTILES = 16
NPS = 25024
SROWS = 25088
STRIPE = SROWS // NTILES
TRASH = NPS
NPAD = NSC * NPS
EPT = E // NTILES
CHUNK = 128
NCHUNK = EPT // CHUNK
TAIL = EPT - NCHUNK * CHUNK
QC = D // 4
ACC = 2 * QC


XROWS = 250
XSPAN = 2000
XWORK = N // XSPAN


def _sc_quarter_x():
    mesh = plsc.VectorSubcoreMesh(core_axis_name="c", subcore_axis_name="s")

    @functools.partial(
        pl.kernel,
        out_type=tuple(
            jax.ShapeDtypeStruct((N, QC), jnp.float32) for _ in range(4)),
        mesh=mesh,
        scratch_types=[
            pltpu.VMEM((XROWS, D), jnp.float32),
        ],
        compiler_params=pltpu.CompilerParams(use_tc_tiling_on_sc=False),
    )
    def k(x_hbm, q0_hbm, q1_hbm, q2_hbm, q3_hbm, buf_v):
        c = lax.axis_index("c")
        s = lax.axis_index("s")
        w = s * NSC + c
        base = w * XSPAN
        oq = [q0_hbm, q1_hbm, q2_hbm, q3_hbm]

        @pl.when(w < XWORK)
        def _():
            @pl.loop(0, XSPAN // XROWS)
            def _(i):
                row0 = base + i * XROWS
                pltpu.sync_copy(x_hbm.at[pl.ds(row0, XROWS)], buf_v)
                for q in range(4):
                    pltpu.sync_copy(
                        buf_v.at[pl.ds(0, XROWS), pl.ds(q * QC, QC)],
                        oq[q].at[pl.ds(row0, XROWS)])

    return k


def _sc_accumulate():
    mesh = plsc.VectorSubcoreMesh(core_axis_name="c", subcore_axis_name="s")

    @functools.partial(
        pl.kernel,
        out_type=tuple(
            jax.ShapeDtypeStruct((NPAD, ACC), jnp.float32) for _ in range(4)),
        mesh=mesh,
        scratch_types=[
            [pltpu.VMEM((CHUNK,), jnp.int32)] * 4,
            [pltpu.VMEM((CHUNK,), jnp.int32)] * 4,
            [pltpu.VMEM((CHUNK,), jnp.int32)] * 4,
            [pltpu.VMEM((CHUNK, QC), jnp.float32)] * 4,
            [pltpu.VMEM((CHUNK, QC), jnp.float32)] * 4,
            [pltpu.VMEM((CHUNK, ACC), jnp.float32)] * 2,
            pltpu.VMEM((STRIPE, ACC), jnp.float32),
            pltpu.VMEM_SHARED((SROWS, ACC), jnp.float32),
            [pltpu.SemaphoreType.DMA] * 4,
            [pltpu.SemaphoreType.DMA] * 4,
            [pltpu.SemaphoreType.DMA] * 4,
            [pltpu.SemaphoreType.DMA] * 2,
        ],
        compiler_params=pltpu.CompilerParams(use_tc_tiling_on_sc=False),
    )
    def k(x0_hbm, x1_hbm, x2_hbm, x3_hbm,
          esrc_hbm, edst_hbm, ea_hbm,
          o0_hbm, o1_hbm, o2_hbm, o3_hbm,
          src_v, dst_v, sidx_v, xg_v, ea_v, vb_v, zb_v,
          acc_sh, sem_i, sem_g, sem_e, sem_s):
        c = lax.axis_index("c")
        s = lax.axis_index("s")
        base_node = c * NPS
        ebase = s * EPT
        xq = [x0_hbm, x1_hbm, x2_hbm, x3_hbm]
        oq = [o0_hbm, o1_hbm, o2_hbm, o3_hbm]

        def issue_idx(j, b):
            base = ebase + j * CHUNK
            pltpu.async_copy(esrc_hbm.at[pl.ds(base, CHUNK)], src_v[b],
                             sem_i[b])
            pltpu.async_copy(edst_hbm.at[pl.ds(base, CHUNK)], dst_v[b],
                             sem_i[b])

        def wait_idx(b):
            pltpu.make_async_copy(esrc_hbm.at[pl.ds(0, CHUNK)], src_v[b],
                                  sem_i[b]).wait()
            pltpu.make_async_copy(edst_hbm.at[pl.ds(0, CHUNK)], dst_v[b],
                                  sem_i[b]).wait()

        def issue_ge(j, b, q):
            base = ebase + j * CHUNK
            pltpu.async_copy(xq[q].at[src_v[b]], xg_v[b], sem_g[b])
            pltpu.async_copy(
                ea_hbm.at[pl.ds(base, CHUNK), pl.ds(q * QC, QC)],
                ea_v[b], sem_e[b])

        def wait_ge(b, q):
            pltpu.make_async_copy(xq[q].at[src_v[b]], xg_v[b],
                                  sem_g[b]).wait()
            pltpu.make_async_copy(
                ea_hbm.at[pl.ds(0, CHUNK), pl.ds(q * QC, QC)],
                ea_v[b], sem_e[b]).wait()

        def wait_scatter(b2, b4):
            pltpu.make_async_copy(vb_v[b2], acc_sh.at[sidx_v[b4]],
                                  sem_s[b2]).wait()

        def chunk_body(j, b, q, wait_sc, do_idx=True, do_ge=True):
            wait_ge(b, q)

            @plsc.parallel_loop(0, CHUNK // 16, unroll=CHUNK // 16)
            def _(t):
                dv = dst_v[b][pl.ds(t * 16, 16)]
                ld = dv - base_node
                ok = (ld >= 0) & (ld < NPS)
                sidx_v[b][pl.ds(t * 16, 16)] = jnp.where(ok, ld, TRASH)
            if do_idx is True:
                issue_idx(j + 4, b)
            elif do_idx is not None:
                @pl.when(do_idx)
                def _():
                    issue_idx(j + 4, b)
            if do_ge is True:
                wait_idx((b + 3) & 3)
                issue_ge(j + 3, (b + 3) & 3, q)
            elif do_ge is not None:
                @pl.when(do_ge)
                def _():
                    wait_idx((b + 3) & 3)
                    issue_ge(j + 3, (b + 3) & 3, q)

            b2 = b & 1

            @plsc.parallel_loop(0, CHUNK, unroll=8)
            def _(r):
                vx = xg_v[b][r, pl.ds(0, 16)]
                ve = ea_v[b][r, pl.ds(0, 16)]
                m = jnp.maximum(vx + ve, 0.0)
                p = jnp.exp(m)
                vb_v[b2][r, pl.ds(0, 16)] = p
                vb_v[b2][r, pl.ds(16, 16)] = m * p

            if wait_sc is True:
                wait_scatter(b2 ^ 1, (b + 3) & 3)
            elif wait_sc is not None:
                @pl.when(wait_sc)
                def _():
                    wait_scatter(b2 ^ 1, (b + 3) & 3)
            pltpu.async_copy(vb_v[b2], acc_sh.at[sidx_v[b]], sem_s[b2],
                             add=True)

        @pl.loop(0, STRIPE)
        def _(i):
            z = jnp.zeros((16,), jnp.float32)
            zb_v[i, pl.ds(0, 16)] = z
            zb_v[i, pl.ds(16, 16)] = z

        nquads = NCHUNK // 4

        for q in range(4):
            pltpu.sync_copy(zb_v, acc_sh.at[pl.ds(s * STRIPE, STRIPE)])
            plsc.subcore_barrier()

            for b in range(3):
                issue_idx(b, b)
            for b in range(3):
                wait_idx(b)
                issue_ge(b, b, q)
            issue_idx(3, 3)

            @pl.loop(0, nquads)
            def _(i):
                for db in range(4):
                    j = 4 * i + db
                    chunk_body(j, db, q,
                               wait_sc=(j >= 1),
                               do_idx=(j + 4 < NCHUNK),
                               do_ge=(j + 3 < NCHUNK))

            chunk_body(NCHUNK - 2, (NCHUNK - 2) & 3, q, wait_sc=True,
                       do_idx=None, do_ge=None)
            chunk_body(NCHUNK - 1, (NCHUNK - 1) & 3, q, wait_sc=True,
                       do_idx=None, do_ge=None)

            tb = NCHUNK & 3
            tbase = ebase + NCHUNK * CHUNK
            pltpu.sync_copy(esrc_hbm.at[pl.ds(tbase, TAIL)],
                            src_v[tb].at[pl.ds(0, TAIL)])
            pltpu.sync_copy(edst_hbm.at[pl.ds(tbase, TAIL)],
                            dst_v[tb].at[pl.ds(0, TAIL)])
            pltpu.async_copy(xq[q].at[src_v[tb]], xg_v[tb], sem_g[tb])
            pltpu.sync_copy(
                ea_hbm.at[pl.ds(tbase, TAIL), pl.ds(q * QC, QC)],
                ea_v[tb].at[pl.ds(0, TAIL)])
            for t in range(TAIL // 16):
                dv = dst_v[tb][pl.ds(t * 16, 16)]
                ld = dv - base_node
                ok = (ld >= 0) & (ld < NPS)
                sidx_v[tb][pl.ds(t * 16, 16)] = jnp.where(ok, ld, TRASH)
            for t in range(TAIL // 16, CHUNK // 16):
                sidx_v[tb][pl.ds(t * 16, 16)] = jnp.full(
                    (16,), TRASH, jnp.int32)
            pltpu.make_async_copy(xq[q].at[src_v[tb]], xg_v[tb],
                                  sem_g[tb]).wait()

            tb2 = tb & 1

            @plsc.parallel_loop(0, CHUNK, unroll=8)
            def _(r):
                vx = xg_v[tb][r, pl.ds(0, 16)]
                ve = ea_v[tb][r, pl.ds(0, 16)]
                m = jnp.maximum(vx + ve, 0.0)
                p = jnp.exp(m)
                vb_v[tb2][r, pl.ds(0, 16)] = p
                vb_v[tb2][r, pl.ds(16, 16)] = m * p

            wait_scatter((NCHUNK - 1) & 1, (NCHUNK - 1) & 3)
            pltpu.sync_copy(vb_v[tb2], acc_sh.at[sidx_v[tb]], add=True)
            plsc.subcore_barrier()

            gbase = base_node + s * STRIPE

            @pl.when(s < NTILES - 1)
            def _():
                pltpu.sync_copy(acc_sh.at[pl.ds(s * STRIPE, STRIPE)],
                                oq[q].at[pl.ds(gbase, STRIPE)])

            last = NPS - (NTILES - 1) * STRIPE

            @pl.when(s == NTILES - 1)
            def _():
                pltpu.sync_copy(acc_sh.at[pl.ds(s * STRIPE, last)],
                                oq[q].at[pl.ds(gbase, last)])

            plsc.subcore_barrier()

    return k


_ROWS_BLK = 3128


def _mlp_body(a0_ref, a1_ref, a2_ref, a3_ref, x_ref, w1_ref, b1_ref,
              g_ref, bt_ref, w2_ref, b2_ref, o_ref):
    quarters = [a0_ref[...], a1_ref[...], a2_ref[...], a3_ref[...]]
    s_acc = jnp.concatenate([a[:, :QC] for a in quarters], axis=1)
    w_acc = jnp.concatenate([a[:, QC:] for a in quarters], axis=1)
    agg = w_acc / (s_acc + 1e-16)
    out = agg + x_ref[...]
    hpre = jnp.dot(out, w1_ref[...],
                   preferred_element_type=jnp.float32) + b1_ref[...]
    mu = jnp.mean(hpre, axis=-1, keepdims=True)
    var = jnp.mean((hpre - mu) ** 2, axis=-1, keepdims=True)
    hn = (hpre - mu) * lax.rsqrt(var + 1e-5) * g_ref[...] + bt_ref[...]
    hr = jnp.maximum(hn, 0.0)
    o_ref[...] = jnp.dot(hr, w2_ref[...],
                         preferred_element_type=jnp.float32) + b2_ref[...]


def _mlp(accs, x, W1, b1, ln_g, ln_b, W2, b2):
    nblk = pl.cdiv(N, _ROWS_BLK)
    return pl.pallas_call(
        _mlp_body,
        out_shape=jax.ShapeDtypeStruct((N, D), jnp.float32),
        grid=(nblk,),
        in_specs=[
            pl.BlockSpec((_ROWS_BLK, ACC), lambda i: (i, 0)),
            pl.BlockSpec((_ROWS_BLK, ACC), lambda i: (i, 0)),
            pl.BlockSpec((_ROWS_BLK, ACC), lambda i: (i, 0)),
            pl.BlockSpec((_ROWS_BLK, ACC), lambda i: (i, 0)),
            pl.BlockSpec((_ROWS_BLK, D), lambda i: (i, 0)),
            pl.BlockSpec((D, H), lambda i: (0, 0)),
            pl.BlockSpec((1, H), lambda i: (0, 0)),
            pl.BlockSpec((1, H), lambda i: (0, 0)),
            pl.BlockSpec((1, H), lambda i: (0, 0)),
            pl.BlockSpec((H, D), lambda i: (0, 0)),
            pl.BlockSpec((1, D), lambda i: (0, 0)),
        ],
        out_specs=pl.BlockSpec((_ROWS_BLK, D), lambda i: (i, 0)),
    )(*accs, x, W1, b1, ln_g, ln_b, W2, b2)


def kernel(x, edge_index, edge_attr, W1, b1, ln_g, ln_b, W2, b2):
    xqs = _sc_quarter_x()(x)
    accs = _sc_accumulate()(*xqs, edge_index[0], edge_index[1], edge_attr)
    accs = [a[:N] for a in accs]
    return _mlp(accs, x,
                W1, b1.reshape(1, H), ln_g.reshape(1, H),
                ln_b.reshape(1, H), W2, b2.reshape(1, D))


# speedup vs baseline: 4.2038x; 1.0275x over previous
import functools

import jax
import jax.numpy as jnp
from jax import lax
from jax.experimental import pallas as pl
from jax.experimental.pallas import tpu as pltpu
from jax.experimental.pallas import tpu_sc as plsc

N = 50000
E = 800000
D = 64
H = 128
EPS = 1e-7

NSC = 2
NTILES = 16
NPS = 25024
SROWS = 25088
STRIPE = SROWS // NTILES
TRASH = NPS
NPAD = NSC * NPS
EPT = E // NTILES
CHUNK = 128
NCHUNK = EPT // CHUNK
TAIL = EPT - NCHUNK * CHUNK
QC = D // 4
ACC = 2 * QC


XROWS = 250
XSPAN = 2000
XWORK = N // XSPAN


def _sc_quarter_x():
    mesh = plsc.VectorSubcoreMesh(core_axis_name="c", subcore_axis_name="s")

    @functools.partial(
        pl.kernel,
        out_type=tuple(
            jax.ShapeDtypeStruct((N, QC), jnp.float32) for _ in range(4)),
        mesh=mesh,
        scratch_types=[
            pltpu.VMEM((XROWS, D), jnp.float32),
        ],
        compiler_params=pltpu.CompilerParams(use_tc_tiling_on_sc=False),
    )
    def k(x_hbm, q0_hbm, q1_hbm, q2_hbm, q3_hbm, buf_v):
        c = lax.axis_index("c")
        s = lax.axis_index("s")
        w = s * NSC + c
        base = w * XSPAN
        oq = [q0_hbm, q1_hbm, q2_hbm, q3_hbm]

        @pl.when(w < XWORK)
        def _():
            @pl.loop(0, XSPAN // XROWS)
            def _(i):
                row0 = base + i * XROWS
                pltpu.sync_copy(x_hbm.at[pl.ds(row0, XROWS)], buf_v)
                for q in range(4):
                    pltpu.sync_copy(
                        buf_v.at[pl.ds(0, XROWS), pl.ds(q * QC, QC)],
                        oq[q].at[pl.ds(row0, XROWS)])

    return k


def _sc_accumulate():
    mesh = plsc.VectorSubcoreMesh(core_axis_name="c", subcore_axis_name="s")

    @functools.partial(
        pl.kernel,
        out_type=tuple(
            jax.ShapeDtypeStruct((NPAD, ACC), jnp.float32) for _ in range(4)),
        mesh=mesh,
        scratch_types=[
            [pltpu.VMEM((CHUNK,), jnp.int32)] * 4,
            [pltpu.VMEM((CHUNK,), jnp.int32)] * 4,
            [pltpu.VMEM((CHUNK,), jnp.int32)] * 4,
            [pltpu.VMEM((CHUNK, QC), jnp.float32)] * 4,
            [pltpu.VMEM((CHUNK, QC), jnp.float32)] * 4,
            [pltpu.VMEM((CHUNK, ACC), jnp.float32)] * 2,
            pltpu.VMEM((STRIPE, ACC), jnp.float32),
            pltpu.VMEM_SHARED((SROWS, ACC), jnp.float32),
            [pltpu.SemaphoreType.DMA] * 4,
            [pltpu.SemaphoreType.DMA] * 4,
            [pltpu.SemaphoreType.DMA] * 4,
            [pltpu.SemaphoreType.DMA] * 2,
        ],
        compiler_params=pltpu.CompilerParams(use_tc_tiling_on_sc=False),
    )
    def k(x0_hbm, x1_hbm, x2_hbm, x3_hbm,
          esrc_hbm, edst_hbm, ea_hbm,
          o0_hbm, o1_hbm, o2_hbm, o3_hbm,
          src_v, dst_v, sidx_v, xg_v, ea_v, vb_v, zb_v,
          acc_sh, sem_i, sem_g, sem_e, sem_s):
        c = lax.axis_index("c")
        s = lax.axis_index("s")
        base_node = c * NPS
        ebase = s * EPT
        xq = [x0_hbm, x1_hbm, x2_hbm, x3_hbm]
        oq = [o0_hbm, o1_hbm, o2_hbm, o3_hbm]

        def issue_idx(j, b):
            base = ebase + j * CHUNK
            pltpu.async_copy(esrc_hbm.at[pl.ds(base, CHUNK)], src_v[b],
                             sem_i[b])
            pltpu.async_copy(edst_hbm.at[pl.ds(base, CHUNK)], dst_v[b],
                             sem_i[b])

        def wait_idx(b):
            pltpu.make_async_copy(esrc_hbm.at[pl.ds(0, CHUNK)], src_v[b],
                                  sem_i[b]).wait()
            pltpu.make_async_copy(edst_hbm.at[pl.ds(0, CHUNK)], dst_v[b],
                                  sem_i[b]).wait()

        def issue_ge(j, b, q):
            base = ebase + j * CHUNK
            pltpu.async_copy(xq[q].at[src_v[b]], xg_v[b], sem_g[b])
            pltpu.async_copy(
                ea_hbm.at[pl.ds(base, CHUNK), pl.ds(q * QC, QC)],
                ea_v[b], sem_e[b])

        def wait_ge(b, q):
            pltpu.make_async_copy(xq[q].at[src_v[b]], xg_v[b],
                                  sem_g[b]).wait()
            pltpu.make_async_copy(
                ea_hbm.at[pl.ds(0, CHUNK), pl.ds(q * QC, QC)],
                ea_v[b], sem_e[b]).wait()

        def wait_scatter(b2, b4):
            pltpu.make_async_copy(vb_v[b2], acc_sh.at[sidx_v[b4]],
                                  sem_s[b2]).wait()

        def chunk_body(j, b, q, wait_sc, do_idx=True, do_ge=True):
            wait_ge(b, q)

            @plsc.parallel_loop(0, CHUNK // 16, unroll=CHUNK // 16)
            def _(t):
                dv = dst_v[b][pl.ds(t * 16, 16)]
                ld = dv - base_node
                ok = (ld >= 0) & (ld < NPS)
                sidx_v[b][pl.ds(t * 16, 16)] = jnp.where(ok, ld, TRASH)
            if do_idx is True:
                issue_idx(j + 4, b)
            elif do_idx is not None:
                @pl.when(do_idx)
                def _():
                    issue_idx(j + 4, b)
            if do_ge is True:
                wait_idx((b + 3) & 3)
                issue_ge(j + 3, (b + 3) & 3, q)
            elif do_ge is not None:
                @pl.when(do_ge)
                def _():
                    wait_idx((b + 3) & 3)
                    issue_ge(j + 3, (b + 3) & 3, q)

            b2 = b & 1

            @plsc.parallel_loop(0, CHUNK, unroll=8)
            def _(r):
                vx = xg_v[b][r, pl.ds(0, 16)]
                ve = ea_v[b][r, pl.ds(0, 16)]
                m = jnp.maximum(vx + ve, 0.0)
                p = jnp.exp(m)
                vb_v[b2][r, pl.ds(0, 16)] = p
                vb_v[b2][r, pl.ds(16, 16)] = m * p

            if wait_sc is True:
                wait_scatter(b2 ^ 1, (b + 3) & 3)
            elif wait_sc is not None:
                @pl.when(wait_sc)
                def _():
                    wait_scatter(b2 ^ 1, (b + 3) & 3)
            pltpu.async_copy(vb_v[b2], acc_sh.at[sidx_v[b]], sem_s[b2],
                             add=True)

        @pl.loop(0, STRIPE)
        def _(i):
            z = jnp.zeros((16,), jnp.float32)
            zb_v[i, pl.ds(0, 16)] = z
            zb_v[i, pl.ds(16, 16)] = z

        nquads = NCHUNK // 4

        for q in range(4):
            pltpu.sync_copy(zb_v, acc_sh.at[pl.ds(s * STRIPE, STRIPE)])
            plsc.subcore_barrier()

            for b in range(3):
                issue_idx(b, b)
            for b in range(3):
                wait_idx(b)
                issue_ge(b, b, q)
            issue_idx(3, 3)

            @pl.loop(0, nquads)
            def _(i):
                for db in range(4):
                    j = 4 * i + db
                    chunk_body(j, db, q,
                               wait_sc=(j >= 1),
                               do_idx=(j + 4 < NCHUNK),
                               do_ge=(j + 3 < NCHUNK))

            chunk_body(NCHUNK - 2, (NCHUNK - 2) & 3, q, wait_sc=True,
                       do_idx=None, do_ge=None)
            chunk_body(NCHUNK - 1, (NCHUNK - 1) & 3, q, wait_sc=True,
                       do_idx=None, do_ge=None)

            tb = NCHUNK & 3
            tbase = ebase + NCHUNK * CHUNK
            pltpu.sync_copy(esrc_hbm.at[pl.ds(tbase, TAIL)],
                            src_v[tb].at[pl.ds(0, TAIL)])
            pltpu.sync_copy(edst_hbm.at[pl.ds(tbase, TAIL)],
                            dst_v[tb].at[pl.ds(0, TAIL)])
            pltpu.async_copy(xq[q].at[src_v[tb]], xg_v[tb], sem_g[tb])
            pltpu.sync_copy(
                ea_hbm.at[pl.ds(tbase, TAIL), pl.ds(q * QC, QC)],
                ea_v[tb].at[pl.ds(0, TAIL)])
            for t in range(TAIL // 16):
                dv = dst_v[tb][pl.ds(t * 16, 16)]
                ld = dv - base_node
                ok = (ld >= 0) & (ld < NPS)
                sidx_v[tb][pl.ds(t * 16, 16)] = jnp.where(ok, ld, TRASH)
            for t in range(TAIL // 16, CHUNK // 16):
                sidx_v[tb][pl.ds(t * 16, 16)] = jnp.full(
                    (16,), TRASH, jnp.int32)
            pltpu.make_async_copy(xq[q].at[src_v[tb]], xg_v[tb],
                                  sem_g[tb]).wait()

            tb2 = tb & 1

            @plsc.parallel_loop(0, CHUNK, unroll=8)
            def _(r):
                vx = xg_v[tb][r, pl.ds(0, 16)]
                ve = ea_v[tb][r, pl.ds(0, 16)]
                m = jnp.maximum(vx + ve, 0.0)
                p = jnp.exp(m)
                vb_v[tb2][r, pl.ds(0, 16)] = p
                vb_v[tb2][r, pl.ds(16, 16)] = m * p

            wait_scatter((NCHUNK - 1) & 1, (NCHUNK - 1) & 3)
            pltpu.sync_copy(vb_v[tb2], acc_sh.at[sidx_v[tb]], add=True)
            plsc.subcore_barrier()

            gbase = base_node + s * STRIPE

            @pl.when(s < NTILES - 1)
            def _():
                pltpu.sync_copy(acc_sh.at[pl.ds(s * STRIPE, STRIPE)],
                                oq[q].at[pl.ds(gbase, STRIPE)])

            last = NPS - (NTILES - 1) * STRIPE

            @pl.when(s == NTILES - 1)
            def _():
                pltpu.sync_copy(acc_sh.at[pl.ds(s * STRIPE, last)],
                                oq[q].at[pl.ds(gbase, last)])

            plsc.subcore_barrier()

    return k


_ROWS_BLK = 3128


def _mlp_body(a0_ref, a1_ref, a2_ref, a3_ref, x_ref, w1_ref, b1_ref,
              g_ref, bt_ref, w2_ref, b2_ref, o_ref):
    quarters = [a0_ref[...], a1_ref[...], a2_ref[...], a3_ref[...]]
    s_acc = jnp.concatenate([a[:, :QC] for a in quarters], axis=1)
    w_acc = jnp.concatenate([a[:, QC:] for a in quarters], axis=1)
    agg = w_acc / (s_acc + 1e-16)
    out = agg + x_ref[...]
    hpre = jnp.dot(out, w1_ref[...],
                   preferred_element_type=jnp.float32) + b1_ref[...]
    mu = jnp.mean(hpre, axis=-1, keepdims=True)
    var = jnp.mean((hpre - mu) ** 2, axis=-1, keepdims=True)
    hn = (hpre - mu) * lax.rsqrt(var + 1e-5) * g_ref[...] + bt_ref[...]
    hr = jnp.maximum(hn, 0.0)
    o_ref[...] = jnp.dot(hr, w2_ref[...],
                         preferred_element_type=jnp.float32) + b2_ref[...]


def _mlp(accs, x, W1, b1, ln_g, ln_b, W2, b2):
    nblk = pl.cdiv(N, _ROWS_BLK)
    return pl.pallas_call(
        _mlp_body,
        out_shape=jax.ShapeDtypeStruct((N, D), jnp.float32),
        grid=(nblk,),
        in_specs=[
            pl.BlockSpec((_ROWS_BLK, ACC), lambda i: (i, 0)),
            pl.BlockSpec((_ROWS_BLK, ACC), lambda i: (i, 0)),
            pl.BlockSpec((_ROWS_BLK, ACC), lambda i: (i, 0)),
            pl.BlockSpec((_ROWS_BLK, ACC), lambda i: (i, 0)),
            pl.BlockSpec((_ROWS_BLK, D), lambda i: (i, 0)),
            pl.BlockSpec((D, H), lambda i: (0, 0)),
            pl.BlockSpec((1, H), lambda i: (0, 0)),
            pl.BlockSpec((1, H), lambda i: (0, 0)),
            pl.BlockSpec((1, H), lambda i: (0, 0)),
            pl.BlockSpec((H, D), lambda i: (0, 0)),
            pl.BlockSpec((1, D), lambda i: (0, 0)),
        ],
        out_specs=pl.BlockSpec((_ROWS_BLK, D), lambda i: (i, 0)),
    )(*accs, x, W1, b1, ln_g, ln_b, W2, b2)


def kernel(x, edge_index, edge_attr, W1, b1, ln_g, ln_b, W2, b2):
    xqs = _sc_quarter_x()(x)
    accs = _sc_accumulate()(*xqs, edge_index[0], edge_index[1], edge_attr)
    return _mlp(accs, x,
                W1, b1.reshape(1, H), ln_g.reshape(1, H),
                ln_b.reshape(1, H), W2, b2.reshape(1, D))


# speedup vs baseline: 4.2497x; 1.0109x over previous
import functools

import jax
import jax.numpy as jnp
from jax import lax
from jax.experimental import pallas as pl
from jax.experimental.pallas import tpu as pltpu
from jax.experimental.pallas import tpu_sc as plsc

N = 50000
E = 800000
D = 64
H = 128
EPS = 1e-7

NSC = 2
NTILES = 16
NPS = 25024
SROWS = 25088
STRIPE = SROWS // NTILES
TRASH = NPS
NPAD = NSC * NPS
EPT = E // NTILES
CHUNK = 128
NCHUNK = EPT // CHUNK
TAIL = EPT - NCHUNK * CHUNK
QC = D // 4
ACC = 2 * QC


XROWS = 250
XSPAN = 2000
XWORK = N // XSPAN


def _sc_quarter_x():
    mesh = plsc.VectorSubcoreMesh(core_axis_name="c", subcore_axis_name="s")

    @functools.partial(
        pl.kernel,
        out_type=tuple(
            jax.ShapeDtypeStruct((N, QC), jnp.float32) for _ in range(4)),
        mesh=mesh,
        scratch_types=[
            pltpu.VMEM((XROWS, D), jnp.float32),
        ],
        compiler_params=pltpu.CompilerParams(use_tc_tiling_on_sc=False),
    )
    def k(x_hbm, q0_hbm, q1_hbm, q2_hbm, q3_hbm, buf_v):
        c = lax.axis_index("c")
        s = lax.axis_index("s")
        w = s * NSC + c
        base = w * XSPAN
        oq = [q0_hbm, q1_hbm, q2_hbm, q3_hbm]

        @pl.when(w < XWORK)
        def _():
            @pl.loop(0, XSPAN // XROWS)
            def _(i):
                row0 = base + i * XROWS
                pltpu.sync_copy(x_hbm.at[pl.ds(row0, XROWS)], buf_v)
                for q in range(4):
                    pltpu.sync_copy(
                        buf_v.at[pl.ds(0, XROWS), pl.ds(q * QC, QC)],
                        oq[q].at[pl.ds(row0, XROWS)])

    return k


def _sc_accumulate():
    mesh = plsc.VectorSubcoreMesh(core_axis_name="c", subcore_axis_name="s")

    @functools.partial(
        pl.kernel,
        out_type=tuple(
            jax.ShapeDtypeStruct((NPAD, ACC), jnp.float32) for _ in range(4)),
        mesh=mesh,
        scratch_types=[
            [pltpu.VMEM((2, CHUNK), jnp.int32)] * 4,
            [pltpu.VMEM((CHUNK,), jnp.int32)] * 4,
            [pltpu.VMEM((CHUNK, QC), jnp.float32)] * 4,
            [pltpu.VMEM((CHUNK, QC), jnp.float32)] * 4,
            [pltpu.VMEM((CHUNK, ACC), jnp.float32)] * 2,
            pltpu.VMEM((STRIPE, ACC), jnp.float32),
            pltpu.VMEM_SHARED((SROWS, ACC), jnp.float32),
            [pltpu.SemaphoreType.DMA] * 4,
            [pltpu.SemaphoreType.DMA] * 4,
            [pltpu.SemaphoreType.DMA] * 4,
            [pltpu.SemaphoreType.DMA] * 2,
        ],
        compiler_params=pltpu.CompilerParams(use_tc_tiling_on_sc=False),
    )
    def k(x0_hbm, x1_hbm, x2_hbm, x3_hbm,
          ei_hbm, ea_hbm,
          o0_hbm, o1_hbm, o2_hbm, o3_hbm,
          eidx_v, sidx_v, xg_v, ea_v, vb_v, zb_v,
          acc_sh, sem_i, sem_g, sem_e, sem_s):
        c = lax.axis_index("c")
        s = lax.axis_index("s")
        base_node = c * NPS
        ebase = s * EPT
        xq = [x0_hbm, x1_hbm, x2_hbm, x3_hbm]
        oq = [o0_hbm, o1_hbm, o2_hbm, o3_hbm]

        def issue_idx(j, b):
            base = ebase + j * CHUNK
            pltpu.async_copy(ei_hbm.at[:, pl.ds(base, CHUNK)], eidx_v[b],
                             sem_i[b])

        def wait_idx(b):
            pltpu.make_async_copy(ei_hbm.at[:, pl.ds(0, CHUNK)], eidx_v[b],
                                  sem_i[b]).wait()

        def issue_ge(j, b, q):
            base = ebase + j * CHUNK
            pltpu.async_copy(xq[q].at[eidx_v[b].at[0]], xg_v[b], sem_g[b])
            pltpu.async_copy(
                ea_hbm.at[pl.ds(base, CHUNK), pl.ds(q * QC, QC)],
                ea_v[b], sem_e[b])

        def wait_ge(b, q):
            pltpu.make_async_copy(xq[q].at[eidx_v[b].at[0]], xg_v[b],
                                  sem_g[b]).wait()
            pltpu.make_async_copy(
                ea_hbm.at[pl.ds(0, CHUNK), pl.ds(q * QC, QC)],
                ea_v[b], sem_e[b]).wait()

        def wait_scatter(b2, b4):
            pltpu.make_async_copy(vb_v[b2], acc_sh.at[sidx_v[b4]],
                                  sem_s[b2]).wait()

        def chunk_body(j, b, q, wait_sc, do_idx=True, do_ge=True):
            wait_ge(b, q)

            @plsc.parallel_loop(0, CHUNK // 16, unroll=CHUNK // 16)
            def _(t):
                dv = eidx_v[b][1, pl.ds(t * 16, 16)]
                ld = dv - base_node
                ok = (ld >= 0) & (ld < NPS)
                sidx_v[b][pl.ds(t * 16, 16)] = jnp.where(ok, ld, TRASH)
            if do_idx is True:
                issue_idx(j + 4, b)
            elif do_idx is not None:
                @pl.when(do_idx)
                def _():
                    issue_idx(j + 4, b)
            if do_ge is True:
                wait_idx((b + 3) & 3)
                issue_ge(j + 3, (b + 3) & 3, q)
            elif do_ge is not None:
                @pl.when(do_ge)
                def _():
                    wait_idx((b + 3) & 3)
                    issue_ge(j + 3, (b + 3) & 3, q)

            b2 = b & 1

            @plsc.parallel_loop(0, CHUNK, unroll=8)
            def _(r):
                vx = xg_v[b][r, pl.ds(0, 16)]
                ve = ea_v[b][r, pl.ds(0, 16)]
                m = jnp.maximum(vx + ve, 0.0)
                p = jnp.exp(m)
                vb_v[b2][r, pl.ds(0, 16)] = p
                vb_v[b2][r, pl.ds(16, 16)] = m * p

            if wait_sc is True:
                wait_scatter(b2 ^ 1, (b + 3) & 3)
            elif wait_sc is not None:
                @pl.when(wait_sc)
                def _():
                    wait_scatter(b2 ^ 1, (b + 3) & 3)
            pltpu.async_copy(vb_v[b2], acc_sh.at[sidx_v[b]], sem_s[b2],
                             add=True)

        @pl.loop(0, STRIPE)
        def _(i):
            z = jnp.zeros((16,), jnp.float32)
            zb_v[i, pl.ds(0, 16)] = z
            zb_v[i, pl.ds(16, 16)] = z

        nquads = NCHUNK // 4

        for q in range(4):
            pltpu.sync_copy(zb_v, acc_sh.at[pl.ds(s * STRIPE, STRIPE)])
            plsc.subcore_barrier()

            for b in range(3):
                issue_idx(b, b)
            for b in range(3):
                wait_idx(b)
                issue_ge(b, b, q)
            issue_idx(3, 3)

            @pl.loop(0, nquads)
            def _(i):
                for db in range(4):
                    j = 4 * i + db
                    chunk_body(j, db, q,
                               wait_sc=(j >= 1),
                               do_idx=(j + 4 < NCHUNK),
                               do_ge=(j + 3 < NCHUNK))

            chunk_body(NCHUNK - 2, (NCHUNK - 2) & 3, q, wait_sc=True,
                       do_idx=None, do_ge=None)
            chunk_body(NCHUNK - 1, (NCHUNK - 1) & 3, q, wait_sc=True,
                       do_idx=None, do_ge=None)

            tb = NCHUNK & 3
            tbase = ebase + NCHUNK * CHUNK
            pltpu.sync_copy(ei_hbm.at[:, pl.ds(tbase, TAIL)],
                            eidx_v[tb].at[:, pl.ds(0, TAIL)])
            pltpu.async_copy(xq[q].at[eidx_v[tb].at[0]], xg_v[tb],
                             sem_g[tb])
            pltpu.sync_copy(
                ea_hbm.at[pl.ds(tbase, TAIL), pl.ds(q * QC, QC)],
                ea_v[tb].at[pl.ds(0, TAIL)])
            for t in range(TAIL // 16):
                dv = eidx_v[tb][1, pl.ds(t * 16, 16)]
                ld = dv - base_node
                ok = (ld >= 0) & (ld < NPS)
                sidx_v[tb][pl.ds(t * 16, 16)] = jnp.where(ok, ld, TRASH)
            for t in range(TAIL // 16, CHUNK // 16):
                sidx_v[tb][pl.ds(t * 16, 16)] = jnp.full(
                    (16,), TRASH, jnp.int32)
            pltpu.make_async_copy(xq[q].at[eidx_v[tb].at[0]], xg_v[tb],
                                  sem_g[tb]).wait()

            tb2 = tb & 1

            @plsc.parallel_loop(0, CHUNK, unroll=8)
            def _(r):
                vx = xg_v[tb][r, pl.ds(0, 16)]
                ve = ea_v[tb][r, pl.ds(0, 16)]
                m = jnp.maximum(vx + ve, 0.0)
                p = jnp.exp(m)
                vb_v[tb2][r, pl.ds(0, 16)] = p
                vb_v[tb2][r, pl.ds(16, 16)] = m * p

            wait_scatter((NCHUNK - 1) & 1, (NCHUNK - 1) & 3)
            pltpu.sync_copy(vb_v[tb2], acc_sh.at[sidx_v[tb]], add=True)
            plsc.subcore_barrier()

            gbase = base_node + s * STRIPE

            @pl.when(s < NTILES - 1)
            def _():
                pltpu.sync_copy(acc_sh.at[pl.ds(s * STRIPE, STRIPE)],
                                oq[q].at[pl.ds(gbase, STRIPE)])

            last = NPS - (NTILES - 1) * STRIPE

            @pl.when(s == NTILES - 1)
            def _():
                pltpu.sync_copy(acc_sh.at[pl.ds(s * STRIPE, last)],
                                oq[q].at[pl.ds(gbase, last)])

            plsc.subcore_barrier()

    return k


_ROWS_BLK = 6256


def _mlp_body(a0_ref, a1_ref, a2_ref, a3_ref, x_ref, w1_ref, b1_ref,
              g_ref, bt_ref, w2_ref, b2_ref, o_ref):
    quarters = [a0_ref[...], a1_ref[...], a2_ref[...], a3_ref[...]]
    s_acc = jnp.concatenate([a[:, :QC] for a in quarters], axis=1)
    w_acc = jnp.concatenate([a[:, QC:] for a in quarters], axis=1)
    agg = w_acc / (s_acc + 1e-16)
    out = agg + x_ref[...]
    hpre = jnp.dot(out, w1_ref[...],
                   preferred_element_type=jnp.float32) + b1_ref[...]
    mu = jnp.mean(hpre, axis=-1, keepdims=True)
    var = jnp.mean((hpre - mu) ** 2, axis=-1, keepdims=True)
    hn = (hpre - mu) * lax.rsqrt(var + 1e-5) * g_ref[...] + bt_ref[...]
    hr = jnp.maximum(hn, 0.0)
    o_ref[...] = jnp.dot(hr, w2_ref[...],
                         preferred_element_type=jnp.float32) + b2_ref[...]


def _mlp(accs, x, W1, b1, ln_g, ln_b, W2, b2):
    nblk = pl.cdiv(N, _ROWS_BLK)
    return pl.pallas_call(
        _mlp_body,
        out_shape=jax.ShapeDtypeStruct((N, D), jnp.float32),
        grid=(nblk,),
        in_specs=[
            pl.BlockSpec((_ROWS_BLK, ACC), lambda i: (i, 0)),
            pl.BlockSpec((_ROWS_BLK, ACC), lambda i: (i, 0)),
            pl.BlockSpec((_ROWS_BLK, ACC), lambda i: (i, 0)),
            pl.BlockSpec((_ROWS_BLK, ACC), lambda i: (i, 0)),
            pl.BlockSpec((_ROWS_BLK, D), lambda i: (i, 0)),
            pl.BlockSpec((D, H), lambda i: (0, 0)),
            pl.BlockSpec((1, H), lambda i: (0, 0)),
            pl.BlockSpec((1, H), lambda i: (0, 0)),
            pl.BlockSpec((1, H), lambda i: (0, 0)),
            pl.BlockSpec((H, D), lambda i: (0, 0)),
            pl.BlockSpec((1, D), lambda i: (0, 0)),
        ],
        out_specs=pl.BlockSpec((_ROWS_BLK, D), lambda i: (i, 0)),
    )(*accs, x, W1, b1, ln_g, ln_b, W2, b2)


def kernel(x, edge_index, edge_attr, W1, b1, ln_g, ln_b, W2, b2):
    xqs = _sc_quarter_x()(x)
    accs = _sc_accumulate()(*xqs, edge_index, edge_attr)
    return _mlp(accs, x,
                W1, b1.reshape(1, H), ln_g.reshape(1, H),
                ln_b.reshape(1, H), W2, b2.reshape(1, D))
